# Initial kernel scaffold; baseline (speedup 1.0000x reference)
import jax
import jax.numpy as jnp
from jax.experimental import pallas as pl


def kernel(x, edge_index, edge_type, pairs, w1, root1, b1, w2, root2, b2, W_dec, b_dec):
    raise NotImplementedError("write your pallas kernel here")


# speedup vs baseline: 1.0528x; 1.0528x over previous
import functools
import jax
import jax.numpy as jnp
from jax import lax
from jax.experimental import pallas as pl
from jax.experimental.pallas import tpu as pltpu
from jax.experimental.pallas import tpu_sc as plsc

_NC, _NS = 2, 16
_NW = _NC * _NS


def _sc_gather(table, idx, chunk):
    B = idx.shape[0]
    D = table.shape[1]
    per_w = B // _NW
    n_iter = per_w // chunk
    mesh = plsc.VectorSubcoreMesh(core_axis_name="c", subcore_axis_name="s")

    @functools.partial(
        pl.kernel, mesh=mesh,
        out_type=jax.ShapeDtypeStruct((B, D), jnp.float32),
        scratch_types=[
            pltpu.VMEM((chunk,), jnp.int32),
            pltpu.VMEM((chunk, D), jnp.float32),
            pltpu.SemaphoreType.DMA,
        ],
    )
    def k(table_hbm, idx_hbm, out_hbm, idx_v, rows_v, sem):
        wid = lax.axis_index("s") * _NC + lax.axis_index("c")
        base = wid * per_w

        def body(i, carry):
            off = base + i * chunk
            pltpu.sync_copy(idx_hbm.at[pl.ds(off, chunk)], idx_v)
            pltpu.async_copy(table_hbm.at[idx_v], rows_v, sem).wait()
            pltpu.sync_copy(rows_v, out_hbm.at[pl.ds(off, chunk)])
            return carry

        lax.fori_loop(0, n_iter, body, 0)

    return k(table, idx)


def _counts_inv(dst2, typ2, n_pad, nb, eb):
    e = dst2.shape[1]
    grid = (n_pad // nb, e // eb)

    def kern(dst_ref, typ_ref, out_ref):
        j = pl.program_id(1)

        @pl.when(j == 0)
        def _():
            out_ref[...] = jnp.zeros_like(out_ref)

        rows = lax.broadcasted_iota(jnp.int32, (nb, eb), 0) + pl.program_id(0) * nb
        oh_dst = (rows == dst_ref[...]).astype(jnp.float32)
        rcols = lax.broadcasted_iota(jnp.int32, (eb, 128), 1)
        oh_typ = (rcols == typ_ref[...].reshape(eb, 1)).astype(jnp.float32)
        out_ref[...] += jnp.dot(oh_dst, oh_typ,
                                preferred_element_type=jnp.float32)

        @pl.when(j == grid[1] - 1)
        def _():
            out_ref[...] = 1.0 / jnp.maximum(out_ref[...], 1.0)

    return pl.pallas_call(
        kern,
        grid=grid,
        in_specs=[
            pl.BlockSpec((1, eb), lambda i, j: (0, j)),
            pl.BlockSpec((1, eb), lambda i, j: (0, j)),
        ],
        out_specs=pl.BlockSpec((nb, 128), lambda i, j: (i, 0)),
        out_shape=jax.ShapeDtypeStruct((n_pad, 128), jnp.float32),
    )(dst2, typ2)


def _rel_transform(x, w, nb):
    n, d = x.shape
    r = w.shape[0]
    grid = (r, n // nb)

    def kern(x_ref, w_ref, out_ref):
        out_ref[0] = jnp.dot(x_ref[...], w_ref[0],
                             preferred_element_type=jnp.float32)

    return pl.pallas_call(
        kern,
        grid=grid,
        in_specs=[
            pl.BlockSpec((nb, d), lambda i, j: (j, 0)),
            pl.BlockSpec((1, d, d), lambda i, j: (i, 0, 0)),
        ],
        out_specs=pl.BlockSpec((1, nb, d), lambda i, j: (i, j, 0)),
        out_shape=jax.ShapeDtypeStruct((r, n, d), jnp.float32),
    )(x, w)


def _aggregate(msgs, scale, dst2, xpad, root, bias2, relu, nb, eb):
    e, d = msgs.shape
    n_pad = xpad.shape[0]
    grid = (n_pad // nb, e // eb)

    def kern(dst_ref, msg_ref, scl_ref, x_ref, root_ref, b_ref, out_ref):
        j = pl.program_id(1)

        @pl.when(j == 0)
        def _():
            out_ref[...] = jnp.zeros_like(out_ref)

        rows = lax.broadcasted_iota(jnp.int32, (nb, eb), 0) + pl.program_id(0) * nb
        oh = (rows == dst_ref[...]).astype(jnp.float32)
        m = msg_ref[...] * scl_ref[:, 0:1]
        out_ref[...] += jnp.dot(oh, m, preferred_element_type=jnp.float32)

        @pl.when(j == grid[1] - 1)
        def _():
            o = (out_ref[...] + b_ref[...]
                 + jnp.dot(x_ref[...], root_ref[...],
                           preferred_element_type=jnp.float32))
            out_ref[...] = jnp.maximum(o, 0.0) if relu else o

    return pl.pallas_call(
        kern,
        grid=grid,
        in_specs=[
            pl.BlockSpec((1, eb), lambda i, j: (0, j)),
            pl.BlockSpec((eb, d), lambda i, j: (j, 0)),
            pl.BlockSpec((eb, 128), lambda i, j: (j, 0)),
            pl.BlockSpec((nb, d), lambda i, j: (i, 0)),
            pl.BlockSpec((d, d), lambda i, j: (0, 0)),
            pl.BlockSpec((1, d), lambda i, j: (0, 0)),
        ],
        out_specs=pl.BlockSpec((nb, d), lambda i, j: (i, 0)),
        out_shape=jax.ShapeDtypeStruct((n_pad, d), jnp.float32),
    )(dst2, msgs, scale, xpad, root, bias2)


def _decode(hs, hd, w_top, w_bot, bias2, pb):
    p, d = hs.shape

    def kern(a_ref, b_ref, wt_ref, wb_ref, bb_ref, out_ref):
        out_ref[...] = (jnp.dot(a_ref[...], wt_ref[...],
                                preferred_element_type=jnp.float32)
                        + jnp.dot(b_ref[...], wb_ref[...],
                                  preferred_element_type=jnp.float32)
                        + bb_ref[...])

    return pl.pallas_call(
        kern,
        grid=(p // pb,),
        in_specs=[
            pl.BlockSpec((pb, d), lambda i: (i, 0)),
            pl.BlockSpec((pb, d), lambda i: (i, 0)),
            pl.BlockSpec((d, 128), lambda i: (0, 0)),
            pl.BlockSpec((d, 128), lambda i: (0, 0)),
            pl.BlockSpec((1, 128), lambda i: (0, 0)),
        ],
        out_specs=pl.BlockSpec((pb, 128), lambda i: (i, 0)),
        out_shape=jax.ShapeDtypeStruct((p, 128), jnp.float32),
    )(hs, hd, w_top, w_bot, bias2)


def kernel(x, edge_index, edge_type, pairs, w1, root1, b1, w2, root2, b2,
           W_dec, b_dec):
    n, d = x.shape
    e = edge_index.shape[1]
    r = w1.shape[0]
    p = pairs.shape[0]

    src = edge_index[0].astype(jnp.int32)
    dst = edge_index[1].astype(jnp.int32)
    typ = edge_type.astype(jnp.int32)
    dst2 = dst.reshape(1, e)
    typ2 = typ.reshape(1, e)

    nb, ebk = 1024, 512
    n_pad = ((n + nb - 1) // nb) * nb
    xpad = jnp.pad(x, ((0, n_pad - n), (0, 0)))

    inv = _counts_inv(dst2, typ2, n_pad, nb, ebk)[:n, :r]
    scale_tab = jnp.broadcast_to(inv.reshape(n * r, 1), (n * r, 128))
    scale_tab = jnp.asarray(scale_tab)
    scale_e = _sc_gather(scale_tab, dst * r + typ, 80)

    idx_msg = typ * n + src

    z1 = _rel_transform(x, w1, 2000).reshape(r * n, d)
    m1 = _sc_gather(z1, idx_msg, 80)
    h1 = _aggregate(m1, scale_e, dst2, xpad, root1, b1.reshape(1, d),
                    True, nb, ebk)[:n]

    z2 = _rel_transform(h1, w2, 2000).reshape(r * n, d)
    m2 = _sc_gather(z2, idx_msg, 80)
    h2 = _aggregate(m2, scale_e, dst2,
                    jnp.pad(h1, ((0, n_pad - n), (0, 0))), root2,
                    b2.reshape(1, d), False, nb, ebk)[:n]

    p_pad = ((p + 3199) // 3200) * 3200
    pidx = jnp.concatenate([
        jnp.pad(pairs[:, 0].astype(jnp.int32), (0, p_pad - p)),
        jnp.pad(pairs[:, 1].astype(jnp.int32), (0, p_pad - p)),
    ])
    g = _sc_gather(h2, pidx, 80)
    hs, hd = g[:p], g[p_pad:p_pad + p]

    k_out = W_dec.shape[1]
    wpad = jnp.pad(W_dec, ((0, 0), (0, 128 - k_out)))
    bpad = jnp.pad(b_dec, (0, 128 - k_out)).reshape(1, 128)
    out = _decode(hs, hd, wpad[:d], wpad[d:], bpad, 1000)
    return out[:, :k_out]


# speedup vs baseline: 1.2231x; 1.1617x over previous
import functools
import jax
import jax.numpy as jnp
from jax import lax
from jax.experimental import pallas as pl
from jax.experimental.pallas import tpu as pltpu
from jax.experimental.pallas import tpu_sc as plsc

_NC, _NS = 2, 16
_NW = _NC * _NS


def _sc_gather(table, idx, chunk):
    B = idx.shape[0]
    D = table.shape[1]
    per_w = B // _NW
    n_iter = per_w // chunk
    mesh = plsc.VectorSubcoreMesh(core_axis_name="c", subcore_axis_name="s")

    @functools.partial(
        pl.kernel, mesh=mesh,
        out_type=jax.ShapeDtypeStruct((B, D), jnp.float32),
        scratch_types=[
            pltpu.VMEM((chunk,), jnp.int32),
            pltpu.VMEM((chunk, D), jnp.float32),
            pltpu.SemaphoreType.DMA,
        ],
    )
    def k(table_hbm, idx_hbm, out_hbm, idx_v, rows_v, sem):
        wid = lax.axis_index("s") * _NC + lax.axis_index("c")
        base = wid * per_w

        def body(i, carry):
            off = base + i * chunk
            pltpu.sync_copy(idx_hbm.at[pl.ds(off, chunk)], idx_v)
            pltpu.async_copy(table_hbm.at[idx_v], rows_v, sem).wait()
            pltpu.sync_copy(rows_v, out_hbm.at[pl.ds(off, chunk)])
            return carry

        lax.fori_loop(0, n_iter, body, 0)

    return k(table, idx)


def _counts_inv(dst2, typ2, n_pad, nb, eb):
    e = dst2.shape[1]
    grid = (n_pad // nb, e // eb)

    def kern(dst_ref, typ_ref, out_ref):
        j = pl.program_id(1)

        @pl.when(j == 0)
        def _():
            out_ref[...] = jnp.zeros_like(out_ref)

        rows = lax.broadcasted_iota(jnp.int32, (nb, eb), 0) + pl.program_id(0) * nb
        oh_dst = (rows == dst_ref[...]).astype(jnp.float32)
        rcols = lax.broadcasted_iota(jnp.int32, (eb, 128), 1)
        oh_typ = (rcols == typ_ref[...].reshape(eb, 1)).astype(jnp.float32)
        out_ref[...] += jnp.dot(oh_dst, oh_typ,
                                preferred_element_type=jnp.float32)

        @pl.when(j == grid[1] - 1)
        def _():
            out_ref[...] = 1.0 / jnp.maximum(out_ref[...], 1.0)

    return pl.pallas_call(
        kern,
        grid=grid,
        in_specs=[
            pl.BlockSpec((1, eb), lambda i, j: (0, j)),
            pl.BlockSpec((1, eb), lambda i, j: (0, j)),
        ],
        out_specs=pl.BlockSpec((nb, 128), lambda i, j: (i, 0)),
        out_shape=jax.ShapeDtypeStruct((n_pad, 128), jnp.float32),
    )(dst2, typ2)


def _rel_transform(x, w, nb):
    n, d = x.shape
    r = w.shape[0]
    grid = (r, n // nb)

    def kern(x_ref, w_ref, out_ref):
        out_ref[0] = jnp.dot(x_ref[...], w_ref[0],
                             preferred_element_type=jnp.float32)

    return pl.pallas_call(
        kern,
        grid=grid,
        in_specs=[
            pl.BlockSpec((nb, d), lambda i, j: (j, 0)),
            pl.BlockSpec((1, d, d), lambda i, j: (i, 0, 0)),
        ],
        out_specs=pl.BlockSpec((1, nb, d), lambda i, j: (i, j, 0)),
        out_shape=jax.ShapeDtypeStruct((r, n, d), jnp.float32),
    )(x, w)


def _aggregate(msgs, scale, dst2, xpad, root, bias2, relu, nb, eb):
    e, d = msgs.shape
    n_pad = xpad.shape[0]
    grid = (n_pad // nb, e // eb)

    def kern(dst_ref, msg_ref, scl_ref, x_ref, root_ref, b_ref, out_ref):
        j = pl.program_id(1)

        @pl.when(j == 0)
        def _():
            out_ref[...] = jnp.zeros_like(out_ref)

        rows = lax.broadcasted_iota(jnp.int32, (nb, eb), 0) + pl.program_id(0) * nb
        oh = (rows == dst_ref[...]).astype(jnp.float32)
        m = msg_ref[...] * scl_ref[:, 0:1]
        out_ref[...] += jnp.dot(oh, m, preferred_element_type=jnp.float32)

        @pl.when(j == grid[1] - 1)
        def _():
            o = (out_ref[...] + b_ref[...]
                 + jnp.dot(x_ref[...], root_ref[...],
                           preferred_element_type=jnp.float32))
            out_ref[...] = jnp.maximum(o, 0.0) if relu else o

    return pl.pallas_call(
        kern,
        grid=grid,
        in_specs=[
            pl.BlockSpec((1, eb), lambda i, j: (0, j)),
            pl.BlockSpec((eb, d), lambda i, j: (j, 0)),
            pl.BlockSpec((eb, 128), lambda i, j: (j, 0)),
            pl.BlockSpec((nb, d), lambda i, j: (i, 0)),
            pl.BlockSpec((d, d), lambda i, j: (0, 0)),
            pl.BlockSpec((1, d), lambda i, j: (0, 0)),
        ],
        out_specs=pl.BlockSpec((nb, d), lambda i, j: (i, 0)),
        out_shape=jax.ShapeDtypeStruct((n_pad, d), jnp.float32),
    )(dst2, msgs, scale, xpad, root, bias2)


def _decode(hs, hd, w_top, w_bot, bias2, pb):
    p, d = hs.shape

    def kern(a_ref, b_ref, wt_ref, wb_ref, bb_ref, out_ref):
        out_ref[...] = (jnp.dot(a_ref[...], wt_ref[...],
                                preferred_element_type=jnp.float32)
                        + jnp.dot(b_ref[...], wb_ref[...],
                                  preferred_element_type=jnp.float32)
                        + bb_ref[...])

    return pl.pallas_call(
        kern,
        grid=(p // pb,),
        in_specs=[
            pl.BlockSpec((pb, d), lambda i: (i, 0)),
            pl.BlockSpec((pb, d), lambda i: (i, 0)),
            pl.BlockSpec((d, 128), lambda i: (0, 0)),
            pl.BlockSpec((d, 128), lambda i: (0, 0)),
            pl.BlockSpec((1, 128), lambda i: (0, 0)),
        ],
        out_specs=pl.BlockSpec((pb, 128), lambda i: (i, 0)),
        out_shape=jax.ShapeDtypeStruct((p, 128), jnp.float32),
    )(hs, hd, w_top, w_bot, bias2)


def kernel(x, edge_index, edge_type, pairs, w1, root1, b1, w2, root2, b2,
           W_dec, b_dec):
    n, d = x.shape
    e = edge_index.shape[1]
    r = w1.shape[0]
    p = pairs.shape[0]

    src = edge_index[0].astype(jnp.int32)
    dst = edge_index[1].astype(jnp.int32)
    typ = edge_type.astype(jnp.int32)
    dst2 = dst.reshape(1, e)
    typ2 = typ.reshape(1, e)

    nb, ebk = 1024, 640
    n_pad = ((n + nb - 1) // nb) * nb
    xpad = jnp.pad(x, ((0, n_pad - n), (0, 0)))

    inv = _counts_inv(dst2, typ2, n_pad, nb, ebk)[:n, :r]
    scale_tab = jnp.broadcast_to(inv.reshape(n * r, 1), (n * r, 128))
    scale_tab = jnp.asarray(scale_tab)
    scale_e = _sc_gather(scale_tab, dst * r + typ, 80)

    idx_msg = typ * n + src

    z1 = _rel_transform(x, w1, 2000).reshape(r * n, d)
    m1 = _sc_gather(z1, idx_msg, 80)
    h1 = _aggregate(m1, scale_e, dst2, xpad, root1, b1.reshape(1, d),
                    True, nb, ebk)[:n]

    z2 = _rel_transform(h1, w2, 2000).reshape(r * n, d)
    m2 = _sc_gather(z2, idx_msg, 80)
    h2 = _aggregate(m2, scale_e, dst2,
                    jnp.pad(h1, ((0, n_pad - n), (0, 0))), root2,
                    b2.reshape(1, d), False, nb, ebk)[:n]

    p_pad = ((p + 3199) // 3200) * 3200
    pidx = jnp.concatenate([
        jnp.pad(pairs[:, 0].astype(jnp.int32), (0, p_pad - p)),
        jnp.pad(pairs[:, 1].astype(jnp.int32), (0, p_pad - p)),
    ])
    g = _sc_gather(h2, pidx, 80)
    hs, hd = g[:p], g[p_pad:p_pad + p]

    k_out = W_dec.shape[1]
    wpad = jnp.pad(W_dec, ((0, 0), (0, 128 - k_out)))
    bpad = jnp.pad(b_dec, (0, 128 - k_out)).reshape(1, 128)
    out = _decode(hs, hd, wpad[:d], wpad[d:], bpad, 1000)
    return out[:, :k_out]
